# Initial kernel scaffold; baseline (speedup 1.0000x reference)
import jax
import jax.numpy as jnp
from jax.experimental import pallas as pl


def kernel(x, pos, edge_index, mol_x, batch, W_embed, b_embed, W_msg, W_rbf, W_upd, W_self, W_fc1, b_fc1, W_fc2, b_fc2, W_out, b_out):
    raise NotImplementedError("write your pallas kernel here")


# speedup vs baseline: 3.2639x; 3.2639x over previous
import functools

import jax
import jax.numpy as jnp
import numpy as np
from jax import lax
from jax.experimental import pallas as pl
from jax.experimental.pallas import tpu as pltpu
from jax.experimental.pallas import tpu_sc as plsc

N = 10000
E = 320000
B = 100
D_NODE = 128
D_MSG = 64
N_RADIAL = 32
CUTOFF = 6.0

NW = 32
CH = 128
CPW = 80
E_PAD = NW * CPW * CH
N_PAD = 10240
ROWS_PER_TILE = N_PAD // 16

_MESH = plsc.VectorSubcoreMesh(core_axis_name="c", subcore_axis_name="s")
_SC_PARAMS = pltpu.CompilerParams(use_tc_tiling_on_sc=False)


def _zero_buf(buf):
    @pl.loop(0, CH)
    def _(r):
        for c in range(4):
            buf[r, pl.ds(c * 16, 16)] = jnp.zeros((16,), jnp.float32)


def _sc_gather_body(pos_hbm, srcm, dstm, outs, outd, srcv, dstv, bufs, bufd, sems, semd):
    cid = lax.axis_index("c")
    sid = lax.axis_index("s")
    wid = cid * 16 + sid
    pltpu.sync_copy(srcm.at[pl.ds(wid * CPW, CPW)], srcv)
    pltpu.sync_copy(dstm.at[pl.ds(wid * CPW, CPW)], dstv)
    base = wid * CPW * CH

    @pl.loop(0, CPW)
    def _(j):
        pltpu.async_copy(pos_hbm.at[srcv.at[j]], bufs, sems)
        pltpu.async_copy(pos_hbm.at[dstv.at[j]], bufd, semd)
        pltpu.make_async_copy(pos_hbm.at[srcv.at[j]], bufs, sems).wait()
        pltpu.make_async_copy(pos_hbm.at[dstv.at[j]], bufd, semd).wait()
        pltpu.sync_copy(bufs, outs.at[pl.ds(base + j * CH, CH)])
        pltpu.sync_copy(bufd, outd.at[pl.ds(base + j * CH, CH)])


@jax.jit
def _sc_gather_pos(pos16, srcm, dstm):
    f = pl.kernel(
        _sc_gather_body,
        out_type=(
            jax.ShapeDtypeStruct((E_PAD, 16), jnp.float32),
            jax.ShapeDtypeStruct((E_PAD, 16), jnp.float32),
        ),
        mesh=_MESH,
        scratch_types=[
            pltpu.VMEM((CPW, CH), jnp.int32),
            pltpu.VMEM((CPW, CH), jnp.int32),
            pltpu.VMEM((CH, 16), jnp.float32),
            pltpu.VMEM((CH, 16), jnp.float32),
            pltpu.SemaphoreType.DMA,
            pltpu.SemaphoreType.DMA,
        ],
        compiler_params=_SC_PARAMS,
    )
    return f(pos16, srcm, dstm)


def _sc_conv_body(hm, coeff, srcm, dstm, out, srcv, dstv, g0, g1, c0, c1, acc,
                  gs0, gs1, cs0, cs1):
    cid = lax.axis_index("c")
    sid = lax.axis_index("s")
    wid = cid * 16 + sid
    gbufs = (g0, g1)
    cbufs = (c0, c1)
    gsems = (gs0, gs1)
    csems = (cs0, cs1)

    _zero_buf(g0)
    for k in range(ROWS_PER_TILE // CH):
        pltpu.sync_copy(g0, acc.at[pl.ds(sid * ROWS_PER_TILE + k * CH, CH)])
    plsc.subcore_barrier()

    pltpu.sync_copy(srcm.at[pl.ds(wid * CPW, CPW)], srcv)
    pltpu.sync_copy(dstm.at[pl.ds(wid * CPW, CPW)], dstv)
    ebase = wid * CPW * CH

    def issue(jj, b):
        pltpu.async_copy(hm.at[srcv.at[jj]], gbufs[b], gsems[b])
        pltpu.async_copy(coeff.at[pl.ds(ebase + jj * CH, CH)], cbufs[b], csems[b])

    issue(0, 0)

    @pl.loop(0, CPW, step=2)
    def _(j):
        for b in range(2):
            jj = j + b
            nb = (b + 1) % 2

            @pl.when(jj + 1 < CPW)
            def _():
                issue(jj + 1, nb)

            pltpu.make_async_copy(hm.at[srcv.at[jj]], gbufs[b], gsems[b]).wait()
            pltpu.make_async_copy(coeff.at[pl.ds(ebase + jj * CH, CH)], cbufs[b],
                                  csems[b]).wait()

            gb = gbufs[b]
            cb = cbufs[b]

            @pl.loop(0, CH)
            def _(r):
                for c in range(4):
                    sl = pl.ds(c * 16, 16)
                    gb[r, sl] = gb[r, sl] * cb[r, sl]

            pltpu.sync_copy(gb, acc.at[dstv.at[jj]], add=True)

    plsc.subcore_barrier()
    pltpu.sync_copy(acc.at[pl.ds(sid * ROWS_PER_TILE, ROWS_PER_TILE)],
                    out.at[cid, pl.ds(sid * ROWS_PER_TILE, ROWS_PER_TILE)])


@jax.jit
def _sc_conv(hm, coeff, srcm, dstm):
    f = pl.kernel(
        _sc_conv_body,
        out_type=jax.ShapeDtypeStruct((2, N_PAD, D_MSG), jnp.float32),
        mesh=_MESH,
        scratch_types=[
            pltpu.VMEM((CPW, CH), jnp.int32),
            pltpu.VMEM((CPW, CH), jnp.int32),
            pltpu.VMEM((CH, D_MSG), jnp.float32),
            pltpu.VMEM((CH, D_MSG), jnp.float32),
            pltpu.VMEM((CH, D_MSG), jnp.float32),
            pltpu.VMEM((CH, D_MSG), jnp.float32),
            pltpu.VMEM_SHARED((N_PAD, D_MSG), jnp.float32),
            pltpu.SemaphoreType.DMA,
            pltpu.SemaphoreType.DMA,
            pltpu.SemaphoreType.DMA,
            pltpu.SemaphoreType.DMA,
        ],
        compiler_params=_SC_PARAMS,
    )
    return f(hm, coeff, srcm, dstm)


_EB = 2048
_GAMMA = (N_RADIAL / CUTOFF) ** 2


def _geom_kernel(ps_ref, pd_ref, wr_ref, c0_ref, c1_ref):
    i = pl.program_id(0)
    diff = ps_ref[...] - pd_ref[...]
    ones_b = jnp.full((16, N_RADIAL), 1.0, jnp.float32)
    d2b = jnp.dot(diff * diff, ones_b, preferred_element_type=jnp.float32)
    db = jnp.sqrt(d2b + 1e-8)
    cen = lax.broadcasted_iota(jnp.int32, (1, N_RADIAL), 1).astype(jnp.float32) * (
        CUTOFF / (N_RADIAL - 1))
    rbf = jnp.exp(-_GAMMA * (db - cen) ** 2)
    t = jnp.clip(db * (1.0 / CUTOFF), 0.0, 1.0)
    xx = jnp.float32(np.pi) * (t - 0.5)
    s2 = xx * xx
    p = jnp.float32(-1.0 / 39916800.0)
    p = p * s2 + jnp.float32(1.0 / 362880.0)
    p = p * s2 + jnp.float32(-1.0 / 5040.0)
    p = p * s2 + jnp.float32(1.0 / 120.0)
    p = p * s2 + jnp.float32(-1.0 / 6.0)
    p = p * s2 + jnp.float32(1.0)
    env = 0.5 * (1.0 - xx * p)
    row = lax.broadcasted_iota(jnp.int32, (_EB, N_RADIAL), 0)
    env = jnp.where(i * _EB + row < E, env, 0.0)
    rbe = rbf * env
    c0_ref[...] = jnp.dot(rbe, wr_ref[0], preferred_element_type=jnp.float32)
    c1_ref[...] = jnp.dot(rbe, wr_ref[1], preferred_element_type=jnp.float32)


@jax.jit
def _tc_geom(pos_s, pos_d, W_rbf):
    grid = (E_PAD // _EB,)
    return pl.pallas_call(
        _geom_kernel,
        grid=grid,
        in_specs=[
            pl.BlockSpec((_EB, 16), lambda i: (i, 0)),
            pl.BlockSpec((_EB, 16), lambda i: (i, 0)),
            pl.BlockSpec((2, N_RADIAL, D_MSG), lambda i: (0, 0, 0)),
        ],
        out_specs=[
            pl.BlockSpec((_EB, D_MSG), lambda i: (i, 0)),
            pl.BlockSpec((_EB, D_MSG), lambda i: (i, 0)),
        ],
        out_shape=[
            jax.ShapeDtypeStruct((E_PAD, D_MSG), jnp.float32),
            jax.ShapeDtypeStruct((E_PAD, D_MSG), jnp.float32),
        ],
    )(pos_s, pos_d, W_rbf)


_NB = 512


def _embed_kernel(x_ref, b_ref, mol_ref, wx_ref, wm_ref, be_ref, wmsg_ref,
                  h_ref, hm_ref):
    mb = jnp.dot(mol_ref[...], wm_ref[...], preferred_element_type=jnp.float32)
    col = lax.broadcasted_iota(jnp.int32, (_NB, 128), 1)
    oh = (b_ref[...] == col).astype(jnp.float32)
    h = x_ref[...] @ wx_ref[...] + jnp.dot(oh, mb, preferred_element_type=jnp.float32)
    h = jax.nn.gelu(h + be_ref[...])
    h_ref[...] = h
    hm_ref[...] = jnp.dot(h, wmsg_ref[...], preferred_element_type=jnp.float32)


@jax.jit
def _tc_embed(x_pad, batch_pad, mol_pad, W_x, W_m, b_embed, W_msg0):
    grid = (N_PAD // _NB,)
    return pl.pallas_call(
        _embed_kernel,
        grid=grid,
        in_specs=[
            pl.BlockSpec((_NB, 128), lambda i: (i, 0)),
            pl.BlockSpec((_NB, 1), lambda i: (i, 0)),
            pl.BlockSpec((128, 32), lambda i: (0, 0)),
            pl.BlockSpec((128, 128), lambda i: (0, 0)),
            pl.BlockSpec((32, 128), lambda i: (0, 0)),
            pl.BlockSpec((1, 128), lambda i: (0, 0)),
            pl.BlockSpec((128, 64), lambda i: (0, 0)),
        ],
        out_specs=[
            pl.BlockSpec((_NB, 128), lambda i: (i, 0)),
            pl.BlockSpec((_NB, 64), lambda i: (i, 0)),
        ],
        out_shape=[
            jax.ShapeDtypeStruct((N_PAD, 128), jnp.float32),
            jax.ShapeDtypeStruct((N_PAD, 64), jnp.float32),
        ],
    )(x_pad, batch_pad, mol_pad, W_x, W_m, b_embed, W_msg0)


def _upd_kernel(p0_ref, p1_ref, h_ref, wu_ref, ws_ref, wmsg_ref, ho_ref, hm_ref):
    agg = p0_ref[...] + p1_ref[...]
    h = h_ref[...]
    upd = jnp.dot(agg, wu_ref[...], preferred_element_type=jnp.float32)
    slf = jnp.dot(h, ws_ref[...], preferred_element_type=jnp.float32)
    hn = h + jax.nn.gelu(upd + slf)
    ho_ref[...] = hn
    hm_ref[...] = jnp.dot(hn, wmsg_ref[...], preferred_element_type=jnp.float32)


@jax.jit
def _tc_upd(p0, p1, h, W_upd_l, W_self_l, W_msg_next):
    grid = (N_PAD // _NB,)
    return pl.pallas_call(
        _upd_kernel,
        grid=grid,
        in_specs=[
            pl.BlockSpec((_NB, 64), lambda i: (i, 0)),
            pl.BlockSpec((_NB, 64), lambda i: (i, 0)),
            pl.BlockSpec((_NB, 128), lambda i: (i, 0)),
            pl.BlockSpec((64, 128), lambda i: (0, 0)),
            pl.BlockSpec((128, 128), lambda i: (0, 0)),
            pl.BlockSpec((128, 64), lambda i: (0, 0)),
        ],
        out_specs=[
            pl.BlockSpec((_NB, 128), lambda i: (i, 0)),
            pl.BlockSpec((_NB, 64), lambda i: (i, 0)),
        ],
        out_shape=[
            jax.ShapeDtypeStruct((N_PAD, 128), jnp.float32),
            jax.ShapeDtypeStruct((N_PAD, 64), jnp.float32),
        ],
    )(p0, p1, h, W_upd_l, W_self_l, W_msg_next)


def _final_kernel(p0_ref, p1_ref, h_ref, wu_ref, ws_ref, b_ref, gs_ref, cn_ref):
    i = pl.program_id(0)

    @pl.when(i == 0)
    def _():
        gs_ref[...] = jnp.zeros_like(gs_ref)
        cn_ref[...] = jnp.zeros_like(cn_ref)

    agg = p0_ref[...] + p1_ref[...]
    h = h_ref[...]
    upd = jnp.dot(agg, wu_ref[...], preferred_element_type=jnp.float32)
    slf = jnp.dot(h, ws_ref[...], preferred_element_type=jnp.float32)
    hn = h + jax.nn.gelu(upd + slf)
    col = lax.broadcasted_iota(jnp.int32, (_NB, 128), 1)
    oh = (b_ref[...] == col).astype(jnp.float32)
    dn = (((0,), (0,)), ((), ()))
    gs_ref[...] += lax.dot_general(oh, hn, dn, preferred_element_type=jnp.float32)
    ones = jnp.ones((_NB, 128), jnp.float32)
    cn_ref[...] += lax.dot_general(oh, ones, dn, preferred_element_type=jnp.float32)


@jax.jit
def _tc_final(p0, p1, h, W_upd_l, W_self_l, batch_pad):
    grid = (N_PAD // _NB,)
    return pl.pallas_call(
        _final_kernel,
        grid=grid,
        in_specs=[
            pl.BlockSpec((_NB, 64), lambda i: (i, 0)),
            pl.BlockSpec((_NB, 64), lambda i: (i, 0)),
            pl.BlockSpec((_NB, 128), lambda i: (i, 0)),
            pl.BlockSpec((64, 128), lambda i: (0, 0)),
            pl.BlockSpec((128, 128), lambda i: (0, 0)),
            pl.BlockSpec((_NB, 1), lambda i: (i, 0)),
        ],
        out_specs=[
            pl.BlockSpec((128, 128), lambda i: (0, 0)),
            pl.BlockSpec((128, 128), lambda i: (0, 0)),
        ],
        out_shape=[
            jax.ShapeDtypeStruct((128, 128), jnp.float32),
            jax.ShapeDtypeStruct((128, 128), jnp.float32),
        ],
    )(p0, p1, h, W_upd_l, W_self_l, batch_pad)


def _head_kernel(gs_ref, cn_ref, mol_ref, w1a_ref, w1b_ref, b1_ref, w2_ref,
                 b2_ref, wo_ref, bo_ref, o_ref):
    g = gs_ref[...] / jnp.maximum(cn_ref[...], 1.0)
    z = jnp.dot(g, w1a_ref[...], preferred_element_type=jnp.float32)
    z = z + jnp.dot(mol_ref[...], w1b_ref[...], preferred_element_type=jnp.float32)
    z = jax.nn.gelu(z + b1_ref[...])
    z = jax.nn.gelu(jnp.dot(z, w2_ref[...], preferred_element_type=jnp.float32) + b2_ref[...])
    a = jnp.dot(z, wo_ref[...], preferred_element_type=jnp.float32) + bo_ref[...]
    col = lax.broadcasted_iota(jnp.int32, a.shape, 1)
    o_ref[...] = jnp.where(col == 2, jax.nn.softplus(a), a)


@jax.jit
def _tc_head(gsum, cntm, mol_pad, W1a, W1b, b_fc1, W_fc2, b_fc2, W_outp, b_outp):
    return pl.pallas_call(
        _head_kernel,
        out_shape=jax.ShapeDtypeStruct((128, 128), jnp.float32),
    )(gsum, cntm, mol_pad, W1a, W1b, b_fc1.reshape(1, 128), W_fc2,
      b_fc2.reshape(1, 128), W_outp, b_outp.reshape(1, 128))


def kernel(x, pos, edge_index, mol_x, batch, W_embed, b_embed, W_msg, W_rbf,
           W_upd, W_self, W_fc1, b_fc1, W_fc2, b_fc2, W_out, b_out):
    f32 = jnp.float32
    src = edge_index[0].astype(jnp.int32)
    dst = edge_index[1].astype(jnp.int32)
    srcm = jnp.zeros((E_PAD,), jnp.int32).at[:E].set(src).reshape(E_PAD // CH, CH)
    dstm = jnp.zeros((E_PAD,), jnp.int32).at[:E].set(dst).reshape(E_PAD // CH, CH)
    pos16 = jnp.zeros((N, 16), f32).at[:, :3].set(pos.astype(f32))

    pos_s, pos_d = _sc_gather_pos(pos16, srcm, dstm)
    coeff0, coeff1 = _tc_geom(pos_s, pos_d, W_rbf)

    x_pad = jnp.zeros((N_PAD, 128), f32).at[:N].set(x)
    batch_pad = jnp.full((N_PAD, 1), B, jnp.int32).at[:N, 0].set(batch)
    mol_pad = jnp.zeros((128, 32), f32).at[:B].set(mol_x)

    h0, hm0 = _tc_embed(x_pad, batch_pad, mol_pad, W_embed[:128], W_embed[128:],
                        b_embed.reshape(1, 128), W_msg[0])
    parts0 = _sc_conv(hm0, coeff0, srcm, dstm)
    h1, hm1 = _tc_upd(parts0[0], parts0[1], h0, W_upd[0], W_self[0], W_msg[1])
    parts1 = _sc_conv(hm1, coeff1, srcm, dstm)
    gsum, cntm = _tc_final(parts1[0], parts1[1], h1, W_upd[1], W_self[1], batch_pad)

    W_outp = jnp.zeros((128, 128), f32).at[:, :3].set(W_out)
    b_outp = jnp.zeros((128,), f32).at[:3].set(b_out)
    out = _tc_head(gsum, cntm, mol_pad, W_fc1[:128], W_fc1[128:], b_fc1,
                   W_fc2, b_fc2, W_outp, b_outp)
    return out[:B, :3]
